# baseline (device time: 29409 ns/iter reference)
import jax
import jax.numpy as jnp
from jax import lax
from jax.experimental import pallas as pl
from jax.experimental.pallas import tpu as pltpu

B = 4
S = 512
S_HALF = S // 2
S_QTR = S // 4
K = 512
N = 1024
NJ = 4
N_CH = N // NJ
H = 8
D = 64

CHUNKS = [(b, j) for b in range(B) for j in range(NJ)]


def kernel(O, Wo):
    O_t = jnp.transpose(O, (0, 2, 3, 1))

    def body(o_ref, w_ref, out_ref, send_buf, recv_x, recv_y,
             send_sems_x, recv_sems_x, send_sems_r, recv_sems_y):
        my_x = lax.axis_index("x")
        my_y = lax.axis_index("y")
        xn = (1 - my_x, my_y)
        yn = (my_x, 1 - my_y)

        barrier = pltpu.get_barrier_semaphore()
        for nbr in (xn, yn):
            pl.semaphore_signal(
                barrier, inc=1,
                device_id=nbr, device_id_type=pl.DeviceIdType.MESH,
            )

        wj = [
            w_ref[:, j * N_CH:(j + 1) * N_CH].astype(jnp.bfloat16)
            for j in range(NJ)
        ]

        my_lo = my_x * S_HALF
        send_lo = (1 - my_x) * S_HALF + my_y * S_QTR

        def mm(b, lo, j):
            acc = None
            for h in range(H):
                lhs = o_ref[b, h, :, pl.ds(lo, S_QTR)].astype(jnp.bfloat16)
                part = lax.dot_general(
                    lhs, wj[j][h * D:(h + 1) * D, :],
                    (((0,), (0,)), ((), ())),
                    preferred_element_type=jnp.float32,
                )
                acc = part if acc is None else acc + part
            return acc

        x_rdmas = {}
        first = True
        for b, j in CHUNKS:
            cs = pl.ds(j * N_CH, N_CH)
            send_buf[b, :, cs] = mm(b, send_lo, j).astype(jnp.bfloat16)
            if first:
                pl.semaphore_wait(barrier, 2)
                first = False
            rdma = pltpu.make_async_remote_copy(
                src_ref=send_buf.at[b, :, cs],
                dst_ref=recv_x.at[b, :, cs],
                send_sem=send_sems_x.at[b, j],
                recv_sem=recv_sems_x.at[b, j],
                device_id=xn,
                device_id_type=pl.DeviceIdType.MESH,
            )
            rdma.start()
            x_rdmas[b, j] = rdma

        relays = {}
        for b, j in CHUNKS:
            cs = pl.ds(j * N_CH, N_CH)
            x_rdmas[b, j].wait_recv()
            relay = pltpu.make_async_remote_copy(
                src_ref=recv_x.at[b, :, cs],
                dst_ref=recv_y.at[b, :, cs],
                send_sem=send_sems_r.at[b, j],
                recv_sem=recv_sems_y.at[b, j],
                device_id=yn,
                device_id_type=pl.DeviceIdType.MESH,
            )
            relay.start()
            relays[b, j] = relay
            out_ref[b, pl.ds(my_y * S_QTR, S_QTR), cs] = (
                mm(b, my_lo + my_y * S_QTR, j)
                + recv_x[b, :, cs].astype(jnp.float32)
            ).astype(jnp.bfloat16)

        for b, j in CHUNKS:
            cs = pl.ds(j * N_CH, N_CH)
            own_b = mm(b, my_lo + (1 - my_y) * S_QTR, j)
            relays[b, j].wait_recv()
            out_ref[b, pl.ds((1 - my_y) * S_QTR, S_QTR), cs] = (
                own_b + recv_y[b, :, cs].astype(jnp.float32)
            ).astype(jnp.bfloat16)

        for b, j in CHUNKS:
            x_rdmas[b, j].wait_send()
            relays[b, j].wait_send()

    return pl.pallas_call(
        body,
        out_shape=jax.ShapeDtypeStruct((B, S_HALF, N), jnp.bfloat16),
        in_specs=[
            pl.BlockSpec(memory_space=pltpu.VMEM),
            pl.BlockSpec(memory_space=pltpu.VMEM),
        ],
        out_specs=pl.BlockSpec(memory_space=pltpu.VMEM),
        scratch_shapes=[
            pltpu.VMEM((B, S_QTR, N), jnp.bfloat16),
            pltpu.VMEM((B, S_QTR, N), jnp.bfloat16),
            pltpu.VMEM((B, S_QTR, N), jnp.bfloat16),
            pltpu.SemaphoreType.DMA((B, NJ)),
            pltpu.SemaphoreType.DMA((B, NJ)),
            pltpu.SemaphoreType.DMA((B, NJ)),
            pltpu.SemaphoreType.DMA((B, NJ)),
        ],
        compiler_params=pltpu.CompilerParams(collective_id=0),
    )(O_t, Wo)


# device time: 25800 ns/iter; 1.1399x vs baseline; 1.1399x over previous
import jax
import jax.numpy as jnp
from jax import lax
from jax.experimental import pallas as pl
from jax.experimental.pallas import tpu as pltpu

B = 4
S = 512
S_HALF = S // 2
S_QTR = S // 4
K = 512
N = 1024
NJ = 2
N_CH = N // NJ
H = 8
D = 64

CHUNKS = [(b, j) for b in range(B) for j in range(NJ)]


def kernel(O, Wo):
    O_t = jnp.transpose(O, (0, 2, 3, 1))

    def body(o_ref, w_ref, out_ref, send_buf, recv_x, recv_y,
             send_sems_x, recv_sems_x, send_sems_r, recv_sems_y):
        my_x = lax.axis_index("x")
        my_y = lax.axis_index("y")
        xn = (1 - my_x, my_y)
        yn = (my_x, 1 - my_y)

        barrier = pltpu.get_barrier_semaphore()
        for nbr in (xn, yn):
            pl.semaphore_signal(
                barrier, inc=1,
                device_id=nbr, device_id_type=pl.DeviceIdType.MESH,
            )

        wj = [
            w_ref[:, j * N_CH:(j + 1) * N_CH].astype(jnp.bfloat16)
            for j in range(NJ)
        ]

        my_lo = my_x * S_HALF
        send_lo = (1 - my_x) * S_HALF + my_y * S_QTR

        def mm(b, lo, j):
            acc = None
            for h in range(H):
                lhs = o_ref[b, h, :, pl.ds(lo, S_QTR)].astype(jnp.bfloat16)
                part = lax.dot_general(
                    lhs, wj[j][h * D:(h + 1) * D, :],
                    (((0,), (0,)), ((), ())),
                    preferred_element_type=jnp.float32,
                )
                acc = part if acc is None else acc + part
            return acc

        x_rdmas = {}
        first = True
        for b, j in CHUNKS:
            cs = pl.ds(j * N_CH, N_CH)
            send_buf[b, :, cs] = mm(b, send_lo, j).astype(jnp.bfloat16)
            if first:
                pl.semaphore_wait(barrier, 2)
                first = False
            rdma = pltpu.make_async_remote_copy(
                src_ref=send_buf.at[b, :, cs],
                dst_ref=recv_x.at[b, :, cs],
                send_sem=send_sems_x.at[b, j],
                recv_sem=recv_sems_x.at[b, j],
                device_id=xn,
                device_id_type=pl.DeviceIdType.MESH,
            )
            rdma.start()
            x_rdmas[b, j] = rdma

        relays = {}
        for b, j in CHUNKS:
            cs = pl.ds(j * N_CH, N_CH)
            x_rdmas[b, j].wait_recv()
            relay = pltpu.make_async_remote_copy(
                src_ref=recv_x.at[b, :, cs],
                dst_ref=recv_y.at[b, :, cs],
                send_sem=send_sems_r.at[b, j],
                recv_sem=recv_sems_y.at[b, j],
                device_id=yn,
                device_id_type=pl.DeviceIdType.MESH,
            )
            relay.start()
            relays[b, j] = relay
            out_ref[b, pl.ds(my_y * S_QTR, S_QTR), cs] = (
                mm(b, my_lo + my_y * S_QTR, j)
                + recv_x[b, :, cs].astype(jnp.float32)
            ).astype(jnp.bfloat16)

        for b, j in CHUNKS:
            cs = pl.ds(j * N_CH, N_CH)
            own_b = mm(b, my_lo + (1 - my_y) * S_QTR, j)
            relays[b, j].wait_recv()
            out_ref[b, pl.ds((1 - my_y) * S_QTR, S_QTR), cs] = (
                own_b + recv_y[b, :, cs].astype(jnp.float32)
            ).astype(jnp.bfloat16)

        for b, j in CHUNKS:
            x_rdmas[b, j].wait_send()
            relays[b, j].wait_send()

    return pl.pallas_call(
        body,
        out_shape=jax.ShapeDtypeStruct((B, S_HALF, N), jnp.bfloat16),
        in_specs=[
            pl.BlockSpec(memory_space=pltpu.VMEM),
            pl.BlockSpec(memory_space=pltpu.VMEM),
        ],
        out_specs=pl.BlockSpec(memory_space=pltpu.VMEM),
        scratch_shapes=[
            pltpu.VMEM((B, S_QTR, N), jnp.bfloat16),
            pltpu.VMEM((B, S_QTR, N), jnp.bfloat16),
            pltpu.VMEM((B, S_QTR, N), jnp.bfloat16),
            pltpu.SemaphoreType.DMA((B, NJ)),
            pltpu.SemaphoreType.DMA((B, NJ)),
            pltpu.SemaphoreType.DMA((B, NJ)),
            pltpu.SemaphoreType.DMA((B, NJ)),
        ],
        compiler_params=pltpu.CompilerParams(collective_id=0),
    )(O_t, Wo)
